# baseline (speedup 1.0000x reference)
import functools

import jax
import jax.numpy as jnp
from jax import lax
from jax.experimental import pallas as pl
from jax.experimental.pallas import tpu as pltpu
from jax.experimental.pallas import tpu_sc as plsc

N = 10000
E = 320000
D = 128
ED = 16

NC = 2
NS = 16
NW = NC * NS
EPW = E // NW
CHUNK = 40
NCHUNK = EPW // CHUNK
NPAD = 10240
ROWS_PER_TILE = NPAD // NS


_NB_E = 40
_NB_N = 10
_EROWS = E // _NB_E
_NROWS = N // _NB_N


def _front_body(x_ref, wk_ref, wq_ref, wv_ref, ws_ref,
                bk_ref, bq_ref, bv_ref, bs_ref,
                a_ref, we_ref, be_ref,
                k_ref, qv_ref, init_ref, e_ref):
    e_ref[...] = jnp.dot(a_ref[...], we_ref[...],
                         preferred_element_type=jnp.float32) + be_ref[...]

    @pl.when(pl.program_id(0) < _NB_N)
    def _():
        xb = x_ref[...]
        k_ref[...] = jnp.dot(xb, wk_ref[...], preferred_element_type=jnp.float32) + bk_ref[...]
        qv_ref[:, :D] = jnp.dot(xb, wq_ref[...], preferred_element_type=jnp.float32) + bq_ref[...]
        qv_ref[:, D:] = jnp.dot(xb, wv_ref[...], preferred_element_type=jnp.float32) + bv_ref[...]
        init_ref[0] = jnp.dot(xb, ws_ref[...], preferred_element_type=jnp.float32) + bs_ref[...]
        init_ref[1] = jnp.zeros((_NROWS, D), jnp.float32)


def _front(x, Wk, Wq, Wv, Ws, bk, bq, bv, bias, edge_attr, We, be):
    w_spec = pl.BlockSpec((D, D), lambda i: (0, 0))
    b_spec = pl.BlockSpec((1, D), lambda i: (0, 0))

    def nblk(i):
        return jnp.minimum(i, _NB_N - 1)

    return pl.pallas_call(
        _front_body,
        grid=(_NB_E,),
        in_specs=[pl.BlockSpec((_NROWS, D), lambda i: (nblk(i), 0)),
                  w_spec, w_spec, w_spec, w_spec,
                  b_spec, b_spec, b_spec, b_spec,
                  pl.BlockSpec((_EROWS, ED), lambda i: (i, 0)),
                  pl.BlockSpec((ED, D), lambda i: (0, 0)),
                  b_spec],
        out_specs=[pl.BlockSpec((_NROWS, D), lambda i: (nblk(i), 0)),
                   pl.BlockSpec((_NROWS, 2 * D), lambda i: (nblk(i), 0)),
                   pl.BlockSpec((2, _NROWS, D), lambda i: (0, nblk(i), 0)),
                   pl.BlockSpec((_EROWS, D), lambda i: (i, 0))],
        out_shape=[jax.ShapeDtypeStruct((N, D), jnp.float32),
                   jax.ShapeDtypeStruct((N, 2 * D), jnp.float32),
                   jax.ShapeDtypeStruct((NC, NPAD, D), jnp.float32),
                   jax.ShapeDtypeStruct((E, D), jnp.float32)],
    )(x, Wk, Wq, Wv, Ws,
      bk.reshape(1, D), bq.reshape(1, D), bv.reshape(1, D), bias.reshape(1, D),
      edge_attr, We, be.reshape(1, D))


def _edge_body(kdst_hbm, qv_hbm, e_hbm, src_hbm, dst_hbm, init_hbm,
               parts_hbm,
               src_i, dst_i, kd_v, qv_v, e_v, acc,
               sem_i0, sem_i1, sem_i2, sem_i3,
               sem_g0, sem_g1, sem_e0, sem_e1):
    core = lax.axis_index("c")
    sid = lax.axis_index("s")
    wid = sid * NC + core
    r0 = sid * ROWS_PER_TILE
    ebase = wid * EPW

    pltpu.sync_copy(init_hbm.at[core, pl.ds(r0, ROWS_PER_TILE)],
                    acc.at[pl.ds(r0, ROWS_PER_TILE)])
    plsc.subcore_barrier()

    kd = [kd_v.at[0], kd_v.at[1]]
    qvs = [qv_v.at[0], qv_v.at[1]]
    ev = [e_v.at[0], e_v.at[1]]
    srs = [src_i.at[0], src_i.at[1], src_i.at[2], src_i.at[3]]
    drs = [dst_i.at[0], dst_i.at[1], dst_i.at[2], dst_i.at[3]]
    sg = [sem_g0, sem_g1]
    se = [sem_e0, sem_e1]
    si = [sem_i0, sem_i1, sem_i2, sem_i3]

    def issue_idx(t, islot):
        pltpu.async_copy(src_hbm.at[wid, t], srs[islot], si[islot])
        pltpu.async_copy(dst_hbm.at[wid, t], drs[islot], si[islot])

    def wait_idx(t, islot):
        pltpu.make_async_copy(src_hbm.at[wid, t], srs[islot], si[islot]).wait()
        pltpu.make_async_copy(dst_hbm.at[wid, t], drs[islot], si[islot]).wait()

    def issue(t, dslot, islot):
        pltpu.async_copy(kdst_hbm.at[drs[islot]], kd[dslot], sg[dslot])
        pltpu.async_copy(qv_hbm.at[srs[islot]], qvs[dslot], sg[dslot])
        pltpu.async_copy(e_hbm.at[pl.ds(ebase + t * CHUNK, CHUNK)],
                         ev[dslot], se[dslot])

    def wait(t, dslot, islot):
        pltpu.make_async_copy(kdst_hbm.at[drs[islot]], kd[dslot], sg[dslot]).wait()
        pltpu.make_async_copy(qv_hbm.at[srs[islot]], qvs[dslot], sg[dslot]).wait()
        pltpu.make_async_copy(e_hbm.at[pl.ds(ebase + t * CHUNK, CHUNK)],
                              ev[dslot], se[dslot]).wait()

    def compute_scatter(dslot, islot):
        kd_s, qv_s, e_s = kd[dslot], qvs[dslot], ev[dslot]

        @plsc.parallel_loop(0, CHUNK, step=1, unroll=4)
        def _(i):
            for c in range(D // 16):
                sl = pl.ds(c * 16, 16)
                slv = pl.ds(D + c * 16, 16)
                z = kd_s[i, sl] + qv_s[i, sl] + e_s[i, sl]
                kd_s[i, sl] = qv_s[i, slv] / (1.0 + jnp.exp(-z))

        pltpu.sync_copy(kd_s, acc.at[drs[islot]], add=True)

    issue_idx(0, 0)
    issue_idx(1, 1)
    issue_idx(2, 2)
    issue_idx(3, 3)
    wait_idx(0, 0)
    issue(0, 0, 0)

    def quad_body(u, carry):
        t0 = 4 * u
        wait_idx(t0 + 1, 1)
        issue(t0 + 1, 1, 1)
        wait(t0, 0, 0)
        compute_scatter(0, 0)
        wait_idx(t0 + 2, 2)
        issue(t0 + 2, 0, 2)
        issue_idx(t0 + 4, 0)
        wait(t0 + 1, 1, 1)
        compute_scatter(1, 1)
        wait_idx(t0 + 3, 3)
        issue(t0 + 3, 1, 3)
        issue_idx(t0 + 5, 1)
        wait(t0 + 2, 0, 2)
        compute_scatter(0, 2)
        wait_idx(t0 + 4, 0)
        issue(t0 + 4, 0, 0)
        issue_idx(t0 + 6, 2)
        wait(t0 + 3, 1, 3)
        compute_scatter(1, 3)
        issue_idx(t0 + 7, 3)
        return carry

    nquad = (NCHUNK - 2) // 4
    lax.fori_loop(0, nquad, quad_body, 0)
    t0 = 4 * nquad
    wait_idx(t0 + 1, 1)
    issue(t0 + 1, 1, 1)
    wait(t0, 0, 0)
    compute_scatter(0, 0)
    wait(t0 + 1, 1, 1)
    compute_scatter(1, 1)
    wait_idx(t0 + 2, 2)
    wait_idx(t0 + 3, 3)

    plsc.subcore_barrier()
    pltpu.sync_copy(acc.at[pl.ds(r0, ROWS_PER_TILE)],
                    parts_hbm.at[core, pl.ds(r0, ROWS_PER_TILE)])


NCHUNK_PAD = NCHUNK + 8


def _edge_stage(kdst, qv_t, e, src, dst, init):
    mesh = plsc.VectorSubcoreMesh(core_axis_name="c", subcore_axis_name="s",
                                  num_cores=NC, num_subcores=NS)
    fn = pl.kernel(
        _edge_body,
        out_type=jax.ShapeDtypeStruct((NC, NPAD, D), jnp.float32),
        mesh=mesh,
        scratch_types=[
            pltpu.VMEM((4, CHUNK), jnp.int32),
            pltpu.VMEM((4, CHUNK), jnp.int32),
            pltpu.VMEM((2, CHUNK, D), jnp.float32),
            pltpu.VMEM((2, CHUNK, 2 * D), jnp.float32),
            pltpu.VMEM((2, CHUNK, D), jnp.float32),
            pltpu.VMEM_SHARED((NPAD, D), jnp.float32),
            pltpu.SemaphoreType.DMA,
            pltpu.SemaphoreType.DMA,
            pltpu.SemaphoreType.DMA,
            pltpu.SemaphoreType.DMA,
            pltpu.SemaphoreType.DMA,
            pltpu.SemaphoreType.DMA,
            pltpu.SemaphoreType.DMA,
            pltpu.SemaphoreType.DMA,
        ],
    )
    src_r = jnp.pad(src.reshape(NW, NCHUNK, CHUNK),
                    ((0, 0), (0, NCHUNK_PAD - NCHUNK), (0, 0)))
    dst_r = jnp.pad(dst.reshape(NW, NCHUNK, CHUNK),
                    ((0, 0), (0, NCHUNK_PAD - NCHUNK), (0, 0)))
    return fn(kdst, qv_t, e, src_r, dst_r, init)


def _final_body(p0_ref, p1_ref, o_ref):
    o_ref[...] = jnp.maximum(p0_ref[...] + p1_ref[...], 0.0)


def _final(p0, p1):
    nb = 10
    rows = N // nb
    return pl.pallas_call(
        _final_body,
        grid=(nb,),
        in_specs=[pl.BlockSpec((rows, D), lambda i: (i, 0)),
                  pl.BlockSpec((rows, D), lambda i: (i, 0))],
        out_specs=pl.BlockSpec((rows, D), lambda i: (i, 0)),
        out_shape=jax.ShapeDtypeStruct((N, D), jnp.float32),
    )(p0, p1)


def kernel(x, edge_index, edge_attr, u, batch,
           Wk, bk, Wq, bq, Wv, bv, We, be, Ws, bias):
    src = edge_index[0]
    dst = edge_index[1]
    k_t, qv_t, init, e = _front(x, Wk, Wq, Wv, Ws, bk, bq, bv, bias,
                                edge_attr, We, be)
    parts = _edge_stage(k_t, qv_t, e, src, dst, init)
    out = _final(parts[0, :N], parts[1, :N])
    return (out, edge_attr, u, edge_index)


# speedup vs baseline: 1.0952x; 1.0952x over previous
import jax
import jax.numpy as jnp
from jax import lax
from jax.experimental import pallas as pl
from jax.experimental.pallas import tpu as pltpu
from jax.experimental.pallas import tpu_sc as plsc

N = 10000
E = 320000
D = 128
ED = 16

NC = 2
NS = 16
NW = NC * NS
EPW = E // NW
CHUNK = 40
NCHUNK = EPW // CHUNK
GROUP = NW * CHUNK

NCH_A = 126
NCH_B = NCHUNK - NCH_A
EA_ROWS = NCH_A * GROUP
EB_ROWS = NCH_B * GROUP

NPAD = 10240
ROWS_PER_TILE = NPAD // NS


_NB_A = 21
_NB_B = 31
_NB_N = 10
_EROWS_A = EA_ROWS // _NB_A
_EROWS_B = EB_ROWS // _NB_B
_NROWS = N // _NB_N


def _front_a_body(x_ref, wk_ref, wq_ref, wv_ref, ws_ref,
                  bk_ref, bq_ref, bv_ref, bs_ref,
                  a_ref, we_ref, be_ref,
                  k_ref, q_ref, v_ref, init_ref, e_ref):
    e_ref[...] = jnp.dot(a_ref[...], we_ref[...],
                         preferred_element_type=jnp.float32) + be_ref[...]

    @pl.when(pl.program_id(0) < _NB_N)
    def _():
        xb = x_ref[...]
        k_ref[...] = jnp.dot(xb, wk_ref[...], preferred_element_type=jnp.float32) + bk_ref[...]
        q_ref[...] = jnp.dot(xb, wq_ref[...], preferred_element_type=jnp.float32) + bq_ref[...]
        v_ref[...] = jnp.dot(xb, wv_ref[...], preferred_element_type=jnp.float32) + bv_ref[...]
        init_ref[0] = jnp.dot(xb, ws_ref[...], preferred_element_type=jnp.float32) + bs_ref[...]
        init_ref[1] = jnp.zeros((_NROWS, D), jnp.float32)


def _front_a(x, Wk, Wq, Wv, Ws, bk, bq, bv, bias, edge_attr_a, We, be):
    w_spec = pl.BlockSpec((D, D), lambda i: (0, 0))
    b_spec = pl.BlockSpec((1, D), lambda i: (0, 0))

    def nblk(i):
        return jnp.minimum(i, _NB_N - 1)

    return pl.pallas_call(
        _front_a_body,
        grid=(_NB_A,),
        in_specs=[pl.BlockSpec((_NROWS, D), lambda i: (nblk(i), 0)),
                  w_spec, w_spec, w_spec, w_spec,
                  b_spec, b_spec, b_spec, b_spec,
                  pl.BlockSpec((_EROWS_A, ED), lambda i: (i, 0)),
                  pl.BlockSpec((ED, D), lambda i: (0, 0)),
                  b_spec],
        out_specs=[pl.BlockSpec((_NROWS, D), lambda i: (nblk(i), 0)),
                   pl.BlockSpec((_NROWS, D), lambda i: (nblk(i), 0)),
                   pl.BlockSpec((_NROWS, D), lambda i: (nblk(i), 0)),
                   pl.BlockSpec((2, _NROWS, D), lambda i: (0, nblk(i), 0)),
                   pl.BlockSpec((_EROWS_A, D), lambda i: (i, 0))],
        out_shape=[jax.ShapeDtypeStruct((N, D), jnp.float32),
                   jax.ShapeDtypeStruct((N, D), jnp.float32),
                   jax.ShapeDtypeStruct((N, D), jnp.float32),
                   jax.ShapeDtypeStruct((NC, NPAD, D), jnp.float32),
                   jax.ShapeDtypeStruct((EA_ROWS, D), jnp.float32)],
    )(x, Wk, Wq, Wv, Ws,
      bk.reshape(1, D), bq.reshape(1, D), bv.reshape(1, D), bias.reshape(1, D),
      edge_attr_a, We, be.reshape(1, D))


def _front_b_body(a_ref, we_ref, be_ref, e_ref):
    e_ref[...] = jnp.dot(a_ref[...], we_ref[...],
                         preferred_element_type=jnp.float32) + be_ref[...]


def _front_b(edge_attr_b, We, be):
    return pl.pallas_call(
        _front_b_body,
        grid=(_NB_B,),
        in_specs=[pl.BlockSpec((_EROWS_B, ED), lambda i: (i, 0)),
                  pl.BlockSpec((ED, D), lambda i: (0, 0)),
                  pl.BlockSpec((1, D), lambda i: (0, 0))],
        out_specs=pl.BlockSpec((_EROWS_B, D), lambda i: (i, 0)),
        out_shape=jax.ShapeDtypeStruct((EB_ROWS, D), jnp.float32),
    )(edge_attr_b, We, be.reshape(1, D))


def _make_edge_body(ch0, nchunk):
    nquad = (nchunk - 2) // 4
    rem = (nchunk - 2) % 4

    def _edge_body(kdst_hbm, q_hbm, v_hbm, e_hbm, src_hbm, dst_hbm, init_hbm,
                   parts_hbm,
                   src_i, dst_i, kd_v, q_v, v_v, e_v, acc,
                   sem_i0, sem_i1, sem_i2, sem_i3,
                   sem_g0, sem_g1, sem_e0, sem_e1):
        core = lax.axis_index("c")
        sid = lax.axis_index("s")
        wid = sid * NC + core
        r0 = sid * ROWS_PER_TILE

        pltpu.sync_copy(init_hbm.at[core, pl.ds(r0, ROWS_PER_TILE)],
                        acc.at[pl.ds(r0, ROWS_PER_TILE)])
        plsc.subcore_barrier()

        kd = [kd_v.at[0], kd_v.at[1]]
        qs = [q_v.at[0], q_v.at[1]]
        vs = [v_v.at[0], v_v.at[1]]
        ev = [e_v.at[0], e_v.at[1]]
        srs = [src_i.at[0], src_i.at[1], src_i.at[2], src_i.at[3]]
        drs = [dst_i.at[0], dst_i.at[1], dst_i.at[2], dst_i.at[3]]
        sg = [sem_g0, sem_g1]
        se = [sem_e0, sem_e1]
        si = [sem_i0, sem_i1, sem_i2, sem_i3]

        def issue_idx(t, islot):
            pltpu.async_copy(src_hbm.at[ch0 + t, wid], srs[islot], si[islot])
            pltpu.async_copy(dst_hbm.at[ch0 + t, wid], drs[islot], si[islot])

        def wait_idx(t, islot):
            pltpu.make_async_copy(src_hbm.at[ch0 + t, wid], srs[islot], si[islot]).wait()
            pltpu.make_async_copy(dst_hbm.at[ch0 + t, wid], drs[islot], si[islot]).wait()

        def issue(t, dslot, islot):
            pltpu.async_copy(kdst_hbm.at[drs[islot]], kd[dslot], sg[dslot])
            pltpu.async_copy(q_hbm.at[srs[islot]], qs[dslot], sg[dslot])
            pltpu.async_copy(v_hbm.at[srs[islot]], vs[dslot], sg[dslot])
            pltpu.async_copy(e_hbm.at[pl.ds((t * NW + wid) * CHUNK, CHUNK)],
                             ev[dslot], se[dslot])

        def wait(t, dslot, islot):
            pltpu.make_async_copy(kdst_hbm.at[drs[islot]], kd[dslot], sg[dslot]).wait()
            pltpu.make_async_copy(q_hbm.at[srs[islot]], qs[dslot], sg[dslot]).wait()
            pltpu.make_async_copy(v_hbm.at[srs[islot]], vs[dslot], sg[dslot]).wait()
            pltpu.make_async_copy(e_hbm.at[pl.ds((t * NW + wid) * CHUNK, CHUNK)],
                                  ev[dslot], se[dslot]).wait()

        def compute_scatter(dslot, islot):
            kd_s, q_s, v_s, e_s = kd[dslot], qs[dslot], vs[dslot], ev[dslot]

            @plsc.parallel_loop(0, CHUNK, step=1, unroll=4)
            def _(i):
                for c in range(D // 16):
                    sl = pl.ds(c * 16, 16)
                    z = kd_s[i, sl] + q_s[i, sl] + e_s[i, sl]
                    kd_s[i, sl] = v_s[i, sl] / (1.0 + jnp.exp(-z))

            pltpu.sync_copy(kd_s, acc.at[drs[islot]], add=True)

        issue_idx(0, 0)
        issue_idx(1, 1)
        issue_idx(2, 2)
        issue_idx(3, 3)
        wait_idx(0, 0)
        issue(0, 0, 0)

        def quad_body(u, carry):
            t0 = 4 * u
            wait_idx(t0 + 1, 1)
            issue(t0 + 1, 1, 1)
            wait(t0, 0, 0)
            compute_scatter(0, 0)
            wait_idx(t0 + 2, 2)
            issue(t0 + 2, 0, 2)
            issue_idx(t0 + 4, 0)
            wait(t0 + 1, 1, 1)
            compute_scatter(1, 1)
            wait_idx(t0 + 3, 3)
            issue(t0 + 3, 1, 3)
            issue_idx(t0 + 5, 1)
            wait(t0 + 2, 0, 2)
            compute_scatter(0, 2)
            wait_idx(t0 + 4, 0)
            issue(t0 + 4, 0, 0)
            issue_idx(t0 + 6, 2)
            wait(t0 + 3, 1, 3)
            compute_scatter(1, 3)
            issue_idx(t0 + 7, 3)
            return carry

        lax.fori_loop(0, nquad, quad_body, 0)
        t0 = 4 * nquad
        wait_idx(t0 + 1, 1)
        issue(t0 + 1, 1, 1)
        wait(t0, 0, 0)
        compute_scatter(0, 0)
        wait(t0 + 1, 1, 1)
        compute_scatter(1, 1)
        if rem == 2:
            wait_idx(t0 + 2, 2)
            issue(t0 + 2, 0, 2)
            wait_idx(t0 + 3, 3)
            issue(t0 + 3, 1, 3)
            wait(t0 + 2, 0, 2)
            compute_scatter(0, 2)
            wait(t0 + 3, 1, 3)
            compute_scatter(1, 3)
        else:
            wait_idx(t0 + 2, 2)
            wait_idx(t0 + 3, 3)

        plsc.subcore_barrier()
        pltpu.sync_copy(acc.at[pl.ds(r0, ROWS_PER_TILE)],
                        parts_hbm.at[core, pl.ds(r0, ROWS_PER_TILE)])

    return _edge_body


NCHUNK_PAD = NCHUNK + 8


def _edge_stage(kdst, q_t, v_t, e_seg, src_r, dst_r, init, ch0, nchunk):
    mesh = plsc.VectorSubcoreMesh(core_axis_name="c", subcore_axis_name="s",
                                  num_cores=NC, num_subcores=NS)
    fn = pl.kernel(
        _make_edge_body(ch0, nchunk),
        out_type=jax.ShapeDtypeStruct((NC, NPAD, D), jnp.float32),
        mesh=mesh,
        scratch_types=[
            pltpu.VMEM((4, CHUNK), jnp.int32),
            pltpu.VMEM((4, CHUNK), jnp.int32),
            pltpu.VMEM((2, CHUNK, D), jnp.float32),
            pltpu.VMEM((2, CHUNK, D), jnp.float32),
            pltpu.VMEM((2, CHUNK, D), jnp.float32),
            pltpu.VMEM((2, CHUNK, D), jnp.float32),
            pltpu.VMEM_SHARED((NPAD, D), jnp.float32),
            pltpu.SemaphoreType.DMA,
            pltpu.SemaphoreType.DMA,
            pltpu.SemaphoreType.DMA,
            pltpu.SemaphoreType.DMA,
            pltpu.SemaphoreType.DMA,
            pltpu.SemaphoreType.DMA,
            pltpu.SemaphoreType.DMA,
            pltpu.SemaphoreType.DMA,
        ],
    )
    return fn(kdst, q_t, v_t, e_seg, src_r, dst_r, init)


def _final_body(p0_ref, p1_ref, o_ref):
    o_ref[...] = jnp.maximum(p0_ref[...] + p1_ref[...], 0.0)


def _final(p0, p1):
    nb = 10
    rows = N // nb
    return pl.pallas_call(
        _final_body,
        grid=(nb,),
        in_specs=[pl.BlockSpec((rows, D), lambda i: (i, 0)),
                  pl.BlockSpec((rows, D), lambda i: (i, 0))],
        out_specs=pl.BlockSpec((rows, D), lambda i: (i, 0)),
        out_shape=jax.ShapeDtypeStruct((N, D), jnp.float32),
    )(p0, p1)


def kernel(x, edge_index, edge_attr, u, batch,
           Wk, bk, Wq, bq, Wv, bv, We, be, Ws, bias):
    src = edge_index[0]
    dst = edge_index[1]
    src_r = jnp.pad(src.reshape(NCHUNK, NW, CHUNK),
                    ((0, NCHUNK_PAD - NCHUNK), (0, 0), (0, 0)))
    dst_r = jnp.pad(dst.reshape(NCHUNK, NW, CHUNK),
                    ((0, NCHUNK_PAD - NCHUNK), (0, 0), (0, 0)))
    k_t, q_t, v_t, init, e_a = _front_a(
        x, Wk, Wq, Wv, Ws, bk, bq, bv, bias, edge_attr[:EA_ROWS], We, be)
    e_b = _front_b(edge_attr[EA_ROWS:], We, be)
    parts_a = _edge_stage(k_t, q_t, v_t, e_a, src_r, dst_r, init, 0, NCH_A)
    parts_b = _edge_stage(k_t, q_t, v_t, e_b, src_r, dst_r, parts_a,
                          NCH_A, NCH_B)
    out = _final(parts_b[0, :N], parts_b[1, :N])
    return (out, edge_attr, u, edge_index)
